# Initial kernel scaffold; baseline (speedup 1.0000x reference)
import jax
import jax.numpy as jnp
from jax.experimental import pallas as pl


def kernel(x, W_c, b_c, W_h, b_h, W_s, b_s):
    raise NotImplementedError("write your pallas kernel here")


# speedup vs baseline: 1.6416x; 1.6416x over previous
import jax
import jax.numpy as jnp
from jax.experimental import pallas as pl
from jax.experimental.pallas import tpu as pltpu

_NUM_SLOTS = 64
_MIN_K = 2
_MAX_K = 8
_NEG_BIG = -1000000000.0


def _router_body(n_tokens, xb_ref, wh_ref, wc_ref, ws_ref, bh_ref, bs_ref,
                 bc_ref, idx_ref, w_ref, budget_ref, aux_ref,
                 probs_acc, counts_acc):
    i = pl.program_id(0)
    nsteps = pl.num_programs(0)
    tb = xb_ref.shape[0]

    xb = xb_ref[...]
    h = jnp.dot(xb, wh_ref[...], preferred_element_type=jnp.float32)
    h = jnp.maximum(h + bh_ref[...], 0.0)
    sc = jnp.dot(h.astype(jnp.bfloat16), ws_ref[...],
                 preferred_element_type=jnp.float32) + bs_ref[...]

    xf = xb.astype(jnp.float32)
    wcf = wc_ref[...].astype(jnp.float32)
    c = jnp.sum(xf * wcf, axis=1, keepdims=True) + bc_ref[0]
    s = jax.nn.sigmoid(c)
    budget = jnp.floor(_MIN_K + (_MAX_K - _MIN_K) * s * s).astype(jnp.int32)
    budget_ref[...] = budget

    iota64 = jax.lax.broadcasted_iota(jnp.int32, (tb, _NUM_SLOTS), 1)
    work = sc
    cols_i = []
    cols_v = []
    onehot_acc = jnp.zeros((tb, _NUM_SLOTS), jnp.float32)
    for k in range(_MAX_K):
        m = jnp.max(work, axis=1, keepdims=True)
        am = jnp.min(jnp.where(work == m, iota64, _NUM_SLOTS),
                     axis=1, keepdims=True)
        sel = iota64 == am
        mk = (budget > k).astype(jnp.float32)
        onehot_acc = onehot_acc + sel.astype(jnp.float32) * mk
        work = jnp.where(sel, _NEG_BIG * 2.0, work)
        cols_i.append(am)
        cols_v.append(m)
    idx_ref[...] = jnp.concatenate(cols_i, axis=1)
    v = jnp.concatenate(cols_v, axis=1)

    ranks = jax.lax.broadcasted_iota(jnp.int32, (tb, _MAX_K), 1)
    mask = ranks < budget
    ml = jnp.where(mask, v, _NEG_BIG)
    e = jnp.exp(ml - jnp.max(ml, axis=1, keepdims=True))
    w = e / jnp.sum(e, axis=1, keepdims=True)
    w_ref[...] = w * mask.astype(jnp.float32)

    pe = jnp.exp(sc - jnp.max(sc, axis=1, keepdims=True))
    probs = pe / jnp.sum(pe, axis=1, keepdims=True)

    @pl.when(i == 0)
    def _init():
        probs_acc[...] = jnp.zeros_like(probs_acc)
        counts_acc[...] = jnp.zeros_like(counts_acc)

    probs_acc[...] += jnp.sum(probs, axis=0, keepdims=True)
    counts_acc[...] += jnp.sum(onehot_acc, axis=0, keepdims=True)

    @pl.when(i == nsteps - 1)
    def _fin():
        total = jnp.float32(n_tokens)
        avg_probs = probs_acc[...] / total
        avg_sel = counts_acc[...] / total
        aux = _NUM_SLOTS * jnp.sum(avg_probs * avg_sel)
        aux_ref[...] = jnp.full((1, 1), aux, jnp.float32)


def kernel(x, W_c, b_c, W_h, b_h, W_s, b_s):
    B, S, D = x.shape
    N = B * S
    R = W_h.shape[1]
    TB = 512
    grid = N // TB

    xb = x.reshape(N, D).astype(jnp.bfloat16)
    whb = W_h.astype(jnp.bfloat16)
    wsb = W_s.astype(jnp.bfloat16)
    wcb = W_c.astype(jnp.bfloat16).reshape(1, D)
    bh2 = b_h.reshape(1, R)
    bs2 = b_s.reshape(1, _NUM_SLOTS)

    import functools
    body = functools.partial(_router_body, N)

    idx, w, budget, aux = pl.pallas_call(
        body,
        grid=(grid,),
        in_specs=[
            pl.BlockSpec((TB, D), lambda i: (i, 0)),
            pl.BlockSpec((D, R), lambda i: (0, 0)),
            pl.BlockSpec((1, D), lambda i: (0, 0)),
            pl.BlockSpec((R, _NUM_SLOTS), lambda i: (0, 0)),
            pl.BlockSpec((1, R), lambda i: (0, 0)),
            pl.BlockSpec((1, _NUM_SLOTS), lambda i: (0, 0)),
            pl.BlockSpec(memory_space=pltpu.SMEM),
        ],
        out_specs=[
            pl.BlockSpec((TB, _MAX_K), lambda i: (i, 0)),
            pl.BlockSpec((TB, _MAX_K), lambda i: (i, 0)),
            pl.BlockSpec((TB, 1), lambda i: (i, 0)),
            pl.BlockSpec((1, 1), lambda i: (0, 0)),
        ],
        out_shape=[
            jax.ShapeDtypeStruct((N, _MAX_K), jnp.int32),
            jax.ShapeDtypeStruct((N, _MAX_K), jnp.float32),
            jax.ShapeDtypeStruct((N, 1), jnp.int32),
            jax.ShapeDtypeStruct((1, 1), jnp.float32),
        ],
        scratch_shapes=[
            pltpu.VMEM((1, _NUM_SLOTS), jnp.float32),
            pltpu.VMEM((1, _NUM_SLOTS), jnp.float32),
        ],
        compiler_params=pltpu.CompilerParams(
            dimension_semantics=("arbitrary",),
        ),
    )(xb, whb, wcb, wsb, bh2, bs2, b_c)

    return (idx.reshape(B, S, _MAX_K), w.reshape(B, S, _MAX_K),
            budget.reshape(B, S, 1), aux[0, 0])


# speedup vs baseline: 1.8161x; 1.1063x over previous
import jax
import jax.numpy as jnp
from jax.experimental import pallas as pl
from jax.experimental.pallas import tpu as pltpu

_NUM_SLOTS = 64
_MIN_K = 2
_MAX_K = 8
_NEG_BIG = -1000000000.0


def _router_body(n_tokens, r_dim, x_ref, wcat_ref, ws_ref, bh_ref, bs_ref,
                 bc_ref, idx_ref, w_ref, budget_ref, aux_ref,
                 probs_acc, counts_acc):
    i = pl.program_id(0)
    nsteps = pl.num_programs(0)
    tb = x_ref.shape[0]

    xb = x_ref[...].astype(jnp.bfloat16)
    hfull = jnp.dot(xb, wcat_ref[...], preferred_element_type=jnp.float32)
    h = jnp.maximum(hfull[:, :r_dim] + bh_ref[...], 0.0)
    sc = jnp.dot(h.astype(jnp.bfloat16), ws_ref[...],
                 preferred_element_type=jnp.float32) + bs_ref[...]

    c = hfull[:, r_dim:r_dim + 1] + bc_ref[0]
    s = jax.nn.sigmoid(c)
    budget = jnp.floor(_MIN_K + (_MAX_K - _MIN_K) * s * s).astype(jnp.int32)
    budget_ref[...] = budget

    iota64 = jax.lax.broadcasted_iota(jnp.int32, (tb, _NUM_SLOTS), 1)
    work = sc
    cols_i = []
    cols_v = []
    onehot_acc = jnp.zeros((tb, _NUM_SLOTS), jnp.float32)
    for k in range(_MAX_K):
        m = jnp.max(work, axis=1, keepdims=True)
        am = jnp.min(jnp.where(work == m, iota64, _NUM_SLOTS),
                     axis=1, keepdims=True)
        sel = iota64 == am
        mk = (budget > k).astype(jnp.float32)
        onehot_acc = onehot_acc + sel.astype(jnp.float32) * mk
        work = jnp.where(sel, _NEG_BIG * 2.0, work)
        cols_i.append(am)
        cols_v.append(m)
    idx_ref[...] = jnp.concatenate(cols_i, axis=1)
    v = jnp.concatenate(cols_v, axis=1)

    ranks = jax.lax.broadcasted_iota(jnp.int32, (tb, _MAX_K), 1)
    mask = ranks < budget
    ml = jnp.where(mask, v, _NEG_BIG)
    e = jnp.exp(ml - jnp.max(ml, axis=1, keepdims=True))
    w = e / jnp.sum(e, axis=1, keepdims=True)
    w_ref[...] = w * mask.astype(jnp.float32)

    pe = jnp.exp(sc - jnp.max(sc, axis=1, keepdims=True))
    probs = pe / jnp.sum(pe, axis=1, keepdims=True)

    @pl.when(i == 0)
    def _init():
        probs_acc[...] = jnp.zeros_like(probs_acc)
        counts_acc[...] = jnp.zeros_like(counts_acc)

    probs_acc[...] += jnp.sum(probs, axis=0, keepdims=True)
    counts_acc[...] += jnp.sum(onehot_acc, axis=0, keepdims=True)

    @pl.when(i == nsteps - 1)
    def _fin():
        total = jnp.float32(n_tokens)
        avg_probs = probs_acc[...] / total
        avg_sel = counts_acc[...] / total
        aux = _NUM_SLOTS * jnp.sum(avg_probs * avg_sel)
        aux_ref[...] = jnp.full((1, 1), aux, jnp.float32)


def kernel(x, W_c, b_c, W_h, b_h, W_s, b_s):
    B, S, D = x.shape
    N = B * S
    R = W_h.shape[1]
    TB = 512
    grid = N // TB

    x2 = x.reshape(N, D)
    ncat = R + 128
    wcat = jnp.concatenate(
        [W_h, W_c, jnp.zeros((D, ncat - R - 1), W_h.dtype)],
        axis=1).astype(jnp.bfloat16)
    wsb = W_s.astype(jnp.bfloat16)
    bh2 = b_h.reshape(1, R)
    bs2 = b_s.reshape(1, _NUM_SLOTS)

    import functools
    body = functools.partial(_router_body, N, R)

    idx, w, budget, aux = pl.pallas_call(
        body,
        grid=(grid,),
        in_specs=[
            pl.BlockSpec((TB, D), lambda i: (i, 0)),
            pl.BlockSpec((D, ncat), lambda i: (0, 0)),
            pl.BlockSpec((R, _NUM_SLOTS), lambda i: (0, 0)),
            pl.BlockSpec((1, R), lambda i: (0, 0)),
            pl.BlockSpec((1, _NUM_SLOTS), lambda i: (0, 0)),
            pl.BlockSpec(memory_space=pltpu.SMEM),
        ],
        out_specs=[
            pl.BlockSpec((TB, _MAX_K), lambda i: (i, 0)),
            pl.BlockSpec((TB, _MAX_K), lambda i: (i, 0)),
            pl.BlockSpec((TB, 1), lambda i: (i, 0)),
            pl.BlockSpec((1, 1), lambda i: (0, 0)),
        ],
        out_shape=[
            jax.ShapeDtypeStruct((N, _MAX_K), jnp.int32),
            jax.ShapeDtypeStruct((N, _MAX_K), jnp.float32),
            jax.ShapeDtypeStruct((N, 1), jnp.int32),
            jax.ShapeDtypeStruct((1, 1), jnp.float32),
        ],
        scratch_shapes=[
            pltpu.VMEM((1, _NUM_SLOTS), jnp.float32),
            pltpu.VMEM((1, _NUM_SLOTS), jnp.float32),
        ],
        compiler_params=pltpu.CompilerParams(
            dimension_semantics=("arbitrary",),
        ),
    )(x2, wcat, wsb, bh2, bs2, b_c)

    return (idx.reshape(B, S, _MAX_K), w.reshape(B, S, _MAX_K),
            budget.reshape(B, S, 1), aux[0, 0])


# speedup vs baseline: 2.0670x; 1.1382x over previous
import jax
import jax.numpy as jnp
from jax.experimental import pallas as pl
from jax.experimental.pallas import tpu as pltpu

_NUM_SLOTS = 64
_MIN_K = 2
_MAX_K = 8
_NEG_BIG = -1000000000.0


def _router_body(n_tokens, r_dim, x_ref, wcat_ref, ws_ref, bh_ref, bs_ref,
                 bc_ref, idx_ref, w_ref, budget_ref, aux_ref,
                 probs_acc, counts_acc):
    i = pl.program_id(0)
    nsteps = pl.num_programs(0)
    tb = x_ref.shape[0]

    xb = x_ref[...].astype(jnp.bfloat16)
    hfull = jnp.dot(xb, wcat_ref[...], preferred_element_type=jnp.float32)
    h = jnp.maximum(hfull[:, :r_dim] + bh_ref[...], 0.0)
    sc = jnp.dot(h.astype(jnp.bfloat16), ws_ref[...],
                 preferred_element_type=jnp.float32) + bs_ref[...]

    c = hfull[:, r_dim:r_dim + 1] + bc_ref[0]
    s = jax.nn.sigmoid(c)
    budget = jnp.floor(_MIN_K + (_MAX_K - _MIN_K) * s * s).astype(jnp.int32)
    budget_ref[...] = budget

    iota_f = jax.lax.broadcasted_iota(
        jnp.int32, (tb, _NUM_SLOTS), 1).astype(jnp.float32)
    budget_f = budget.astype(jnp.float32)
    work = sc
    cols_i = []
    cols_v = []
    onehot_acc = jnp.zeros((tb, _NUM_SLOTS), jnp.float32)
    for k in range(_MAX_K):
        m = jnp.max(work, axis=1, keepdims=True)
        amf = jnp.min(jnp.where(work == m, iota_f, jnp.float32(_NUM_SLOTS)),
                      axis=1, keepdims=True)
        sel = (iota_f == amf).astype(jnp.float32)
        mk = (budget_f > k).astype(jnp.float32)
        onehot_acc = onehot_acc + sel * mk
        work = work - sel * jnp.float32(1e30)
        cols_i.append(amf.astype(jnp.int32))
        cols_v.append(m)
    idx_ref[...] = jnp.concatenate(cols_i, axis=1)
    v = jnp.concatenate(cols_v, axis=1)

    ranks = jax.lax.broadcasted_iota(jnp.int32, (tb, _MAX_K), 1)
    mask = ranks < budget
    ml = jnp.where(mask, v, _NEG_BIG)
    e = jnp.exp(ml - jnp.max(ml, axis=1, keepdims=True))
    w = e / jnp.sum(e, axis=1, keepdims=True)
    w_ref[...] = w * mask.astype(jnp.float32)

    pe = jnp.exp(sc - jnp.max(sc, axis=1, keepdims=True))
    probs = pe / jnp.sum(pe, axis=1, keepdims=True)

    @pl.when(i == 0)
    def _init():
        probs_acc[...] = jnp.zeros_like(probs_acc)
        counts_acc[...] = jnp.zeros_like(counts_acc)

    probs_acc[...] += jnp.sum(probs, axis=0, keepdims=True)
    counts_acc[...] += jnp.sum(onehot_acc, axis=0, keepdims=True)

    @pl.when(i == nsteps - 1)
    def _fin():
        total = jnp.float32(n_tokens)
        avg_probs = probs_acc[...] / total
        avg_sel = counts_acc[...] / total
        aux = _NUM_SLOTS * jnp.sum(avg_probs * avg_sel)
        aux_ref[...] = jnp.full((1, 1), aux, jnp.float32)


def kernel(x, W_c, b_c, W_h, b_h, W_s, b_s):
    B, S, D = x.shape
    N = B * S
    R = W_h.shape[1]
    TB = 512
    grid = N // TB

    x2 = x.reshape(N, D)
    ncat = R + 128
    wcat = jnp.concatenate(
        [W_h, W_c, jnp.zeros((D, ncat - R - 1), W_h.dtype)],
        axis=1).astype(jnp.bfloat16)
    wsb = W_s.astype(jnp.bfloat16)
    bh2 = b_h.reshape(1, R)
    bs2 = b_s.reshape(1, _NUM_SLOTS)

    import functools
    body = functools.partial(_router_body, N, R)

    idx, w, budget, aux = pl.pallas_call(
        body,
        grid=(grid,),
        in_specs=[
            pl.BlockSpec((TB, D), lambda i: (i, 0)),
            pl.BlockSpec((D, ncat), lambda i: (0, 0)),
            pl.BlockSpec((R, _NUM_SLOTS), lambda i: (0, 0)),
            pl.BlockSpec((1, R), lambda i: (0, 0)),
            pl.BlockSpec((1, _NUM_SLOTS), lambda i: (0, 0)),
            pl.BlockSpec(memory_space=pltpu.SMEM),
        ],
        out_specs=[
            pl.BlockSpec((TB, _MAX_K), lambda i: (i, 0)),
            pl.BlockSpec((TB, _MAX_K), lambda i: (i, 0)),
            pl.BlockSpec((TB, 1), lambda i: (i, 0)),
            pl.BlockSpec((1, 1), lambda i: (0, 0)),
        ],
        out_shape=[
            jax.ShapeDtypeStruct((N, _MAX_K), jnp.int32),
            jax.ShapeDtypeStruct((N, _MAX_K), jnp.float32),
            jax.ShapeDtypeStruct((N, 1), jnp.int32),
            jax.ShapeDtypeStruct((1, 1), jnp.float32),
        ],
        scratch_shapes=[
            pltpu.VMEM((1, _NUM_SLOTS), jnp.float32),
            pltpu.VMEM((1, _NUM_SLOTS), jnp.float32),
        ],
        compiler_params=pltpu.CompilerParams(
            dimension_semantics=("arbitrary",),
        ),
    )(x2, wcat, wsb, bh2, bs2, b_c)

    return (idx.reshape(B, S, _MAX_K), w.reshape(B, S, _MAX_K),
            budget.reshape(B, S, 1), aux[0, 0])


# speedup vs baseline: 2.2173x; 1.0727x over previous
import jax
import jax.numpy as jnp
from jax.experimental import pallas as pl
from jax.experimental.pallas import tpu as pltpu

_NUM_SLOTS = 64
_MIN_K = 2
_MAX_K = 8
_NEG_BIG = -1000000000.0


def _router_body(n_tokens, r_dim, x_ref, wcat_ref, ws_ref, bh_ref, bs_ref,
                 bc_ref, idx_ref, w_ref, budget_ref, aux_ref,
                 probs_acc, counts_acc):
    i = pl.program_id(0)
    nsteps = pl.num_programs(0)
    tb = x_ref.shape[0]

    xb = x_ref[...].astype(jnp.bfloat16)
    hfull = jnp.dot(xb, wcat_ref[...], preferred_element_type=jnp.float32)
    h = jnp.maximum(hfull[:, :r_dim] + bh_ref[...], 0.0)
    sc = jnp.dot(h.astype(jnp.bfloat16), ws_ref[...],
                 preferred_element_type=jnp.float32) + bs_ref[...]

    c = hfull[:, r_dim:r_dim + 1] + bc_ref[0]
    s = jax.nn.sigmoid(c)
    budget = jnp.floor(_MIN_K + (_MAX_K - _MIN_K) * s * s).astype(jnp.int32)
    budget_ref[...] = budget

    iota_f = jax.lax.broadcasted_iota(
        jnp.int32, (tb, _NUM_SLOTS), 1).astype(jnp.float32)
    work = sc
    cols_i = []
    cols_v = []
    for k in range(_MAX_K):
        m = jnp.max(work, axis=1, keepdims=True)
        amf = jnp.min(jnp.where(work == m, iota_f, jnp.float32(_NUM_SLOTS)),
                      axis=1, keepdims=True)
        sel = (iota_f == amf).astype(jnp.float32)
        work = work - sel * jnp.float32(1e30)
        cols_i.append(amf)
        cols_v.append(m)
    idxf = jnp.concatenate(cols_i, axis=1)
    idx_ref[...] = idxf.astype(jnp.int32)
    v = jnp.concatenate(cols_v, axis=1)

    ranks8 = jax.lax.broadcasted_iota(
        jnp.int32, (tb, _MAX_K), 1).astype(jnp.float32)
    last = (ranks8 == budget.astype(jnp.float32) - 1.0).astype(jnp.float32)
    thr_v = jnp.sum(v * last, axis=1, keepdims=True)
    thr_i = jnp.sum(idxf * last, axis=1, keepdims=True)
    selected = ((sc > thr_v).astype(jnp.float32)
                + (sc == thr_v).astype(jnp.float32)
                * (iota_f <= thr_i).astype(jnp.float32))

    ranks = jax.lax.broadcasted_iota(jnp.int32, (tb, _MAX_K), 1)
    mask = ranks < budget
    ml = jnp.where(mask, v, _NEG_BIG)
    e = jnp.exp(ml - jnp.max(ml, axis=1, keepdims=True))
    w = e / jnp.sum(e, axis=1, keepdims=True)
    w_ref[...] = w * mask.astype(jnp.float32)

    pe = jnp.exp(sc - jnp.max(sc, axis=1, keepdims=True))
    probs = pe / jnp.sum(pe, axis=1, keepdims=True)

    @pl.when(i == 0)
    def _init():
        probs_acc[...] = jnp.zeros_like(probs_acc)
        counts_acc[...] = jnp.zeros_like(counts_acc)

    probs_acc[...] += jnp.sum(probs, axis=0, keepdims=True)
    counts_acc[...] += jnp.sum(selected, axis=0, keepdims=True)

    @pl.when(i == nsteps - 1)
    def _fin():
        total = jnp.float32(n_tokens)
        avg_probs = probs_acc[...] / total
        avg_sel = counts_acc[...] / total
        aux = _NUM_SLOTS * jnp.sum(avg_probs * avg_sel)
        aux_ref[...] = jnp.full((1, 1), aux, jnp.float32)


def kernel(x, W_c, b_c, W_h, b_h, W_s, b_s):
    B, S, D = x.shape
    N = B * S
    R = W_h.shape[1]
    TB = 1024
    grid = N // TB

    x2 = x.reshape(N, D)
    ncat = R + 128
    wcat = jnp.concatenate(
        [W_h, W_c, jnp.zeros((D, ncat - R - 1), W_h.dtype)],
        axis=1).astype(jnp.bfloat16)
    wsb = W_s.astype(jnp.bfloat16)
    bh2 = b_h.reshape(1, R)
    bs2 = b_s.reshape(1, _NUM_SLOTS)

    import functools
    body = functools.partial(_router_body, N, R)

    idx, w, budget, aux = pl.pallas_call(
        body,
        grid=(grid,),
        in_specs=[
            pl.BlockSpec((TB, D), lambda i: (i, 0)),
            pl.BlockSpec((D, ncat), lambda i: (0, 0)),
            pl.BlockSpec((R, _NUM_SLOTS), lambda i: (0, 0)),
            pl.BlockSpec((1, R), lambda i: (0, 0)),
            pl.BlockSpec((1, _NUM_SLOTS), lambda i: (0, 0)),
            pl.BlockSpec(memory_space=pltpu.SMEM),
        ],
        out_specs=[
            pl.BlockSpec((TB, _MAX_K), lambda i: (i, 0)),
            pl.BlockSpec((TB, _MAX_K), lambda i: (i, 0)),
            pl.BlockSpec((TB, 1), lambda i: (i, 0)),
            pl.BlockSpec((1, 1), lambda i: (0, 0)),
        ],
        out_shape=[
            jax.ShapeDtypeStruct((N, _MAX_K), jnp.int32),
            jax.ShapeDtypeStruct((N, _MAX_K), jnp.float32),
            jax.ShapeDtypeStruct((N, 1), jnp.int32),
            jax.ShapeDtypeStruct((1, 1), jnp.float32),
        ],
        scratch_shapes=[
            pltpu.VMEM((1, _NUM_SLOTS), jnp.float32),
            pltpu.VMEM((1, _NUM_SLOTS), jnp.float32),
        ],
        compiler_params=pltpu.CompilerParams(
            dimension_semantics=("arbitrary",),
        ),
    )(x2, wcat, wsb, bh2, bs2, b_c)

    return (idx.reshape(B, S, _MAX_K), w.reshape(B, S, _MAX_K),
            budget.reshape(B, S, 1), aux[0, 0])
